# baseline (speedup 1.0000x reference)
import functools

import jax
import jax.numpy as jnp
from jax import lax
from jax.experimental import pallas as pl
from jax.experimental.pallas import tpu as pltpu
from jax.experimental.pallas import tpu_sc as plsc

DIM = 32
B = 16384
NC = 2
NS = 16
NW = NC * NS
BPW = B // NW
CHUNK = 128
NCHUNK = BPW // CHUNK

_sc_mesh = plsc.VectorSubcoreMesh(core_axis_name="c", subcore_axis_name="s")


@functools.partial(
    pl.kernel,
    out_type=[jax.ShapeDtypeStruct((B, DIM), jnp.float32)] * 4,
    mesh=_sc_mesh,
    scratch_types=(
        [pltpu.VMEM((NCHUNK, CHUNK), jnp.int32) for _ in range(4)]
        + [pltpu.VMEM((BPW, DIM), jnp.float32) for _ in range(4)]
        + [pltpu.SemaphoreType.DMA]
    ),
    compiler_params=pltpu.CompilerParams(use_tc_tiling_on_sc=False),
)
def _sc_gather4(u_idx, i_idx, s_idx, g_idx, ut, it, st, gt,
                ue_o, ie_o, se_o, ge_o,
                u_iv, i_iv, s_iv, g_iv,
                u_rv, i_rv, s_rv, g_rv, sem):
    wid = lax.axis_index("s") * NC + lax.axis_index("c")
    base = wid * BPW
    pltpu.sync_copy(u_idx.at[wid], u_iv)
    pltpu.sync_copy(i_idx.at[wid], i_iv)
    pltpu.sync_copy(s_idx.at[wid], s_iv)
    pltpu.sync_copy(g_idx.at[wid], g_iv)
    descs = []
    for iv, rv, tab in ((u_iv, u_rv, ut), (i_iv, i_rv, it),
                        (s_iv, s_rv, st), (g_iv, g_rv, gt)):
        for j in range(NCHUNK):
            descs.append(
                pltpu.async_copy(tab.at[iv.at[j]],
                                 rv.at[pl.ds(j * CHUNK, CHUNK)], sem))
    for d in descs:
        d.wait()
    pltpu.sync_copy(u_rv, ue_o.at[pl.ds(base, BPW)])
    pltpu.sync_copy(i_rv, ie_o.at[pl.ds(base, BPW)])
    pltpu.sync_copy(s_rv, se_o.at[pl.ds(base, BPW)])
    pltpu.sync_copy(g_rv, ge_o.at[pl.ds(base, BPW)])


TB = 2048


def _mlp_body(ue, ie, se, ge, w1u, w1i, w1s, w1g, b1, w2, b2, w3, b3, w4, b4,
              wo, bo, out):
    h = (jnp.dot(ue[...], w1u[...], preferred_element_type=jnp.float32)
         + jnp.dot(ie[...], w1i[...], preferred_element_type=jnp.float32)
         + jnp.dot(se[...], w1s[...], preferred_element_type=jnp.float32)
         + jnp.dot(ge[...], w1g[...], preferred_element_type=jnp.float32)
         + b1[...])
    h = jnp.maximum(h, 0.0)
    h = jnp.maximum(jnp.dot(h, w2[...], preferred_element_type=jnp.float32)
                    + b2[...], 0.0)
    h = jnp.maximum(jnp.dot(h, w3[...], preferred_element_type=jnp.float32)
                    + b3[...], 0.0)
    h = jnp.maximum(jnp.dot(h, w4[...], preferred_element_type=jnp.float32)
                    + b4[...], 0.0)
    out[...] = jnp.dot(h, wo[...], preferred_element_type=jnp.float32) + bo[...]


def _mlp(ue, ie, se, ge, W1, b1, W2, b2, W3, b3, W4, b4, Wo, bo):
    w1t = W1.T
    full = lambda shape: pl.BlockSpec(shape, lambda i: (0, 0))
    emb = pl.BlockSpec((TB, DIM), lambda i: (i, 0))
    return pl.pallas_call(
        _mlp_body,
        grid=(B // TB,),
        in_specs=[emb, emb, emb, emb,
                  full((DIM, 64)), full((DIM, 64)), full((DIM, 64)),
                  full((DIM, 64)), full((1, 64)),
                  full((64, 32)), full((1, 32)),
                  full((32, 16)), full((1, 16)),
                  full((16, 8)), full((1, 8)),
                  full((8, 1)), full((1, 1))],
        out_specs=pl.BlockSpec((TB, 1), lambda i: (i, 0)),
        out_shape=jax.ShapeDtypeStruct((B, 1), jnp.float32),
        compiler_params=pltpu.CompilerParams(
            dimension_semantics=("arbitrary",)),
    )(ue, ie, se, ge,
      w1t[0:DIM], w1t[DIM:2 * DIM], w1t[2 * DIM:3 * DIM], w1t[3 * DIM:],
      b1.reshape(1, 64), W2.T, b2.reshape(1, 32), W3.T, b3.reshape(1, 16),
      W4.T, b4.reshape(1, 8), Wo.T, bo.reshape(1, 1))


def kernel(user_indices, item_indices, social_indices, giver_indices,
           user_table, item_table, social_table, giver_table,
           W1, b1, W2, b2, W3, b3, W4, b4, Wo, bo):
    ui = jnp.clip(user_indices, 0, user_table.shape[0] - 1)
    ii = jnp.clip(item_indices, 0, item_table.shape[0] - 1)
    si = jnp.clip(social_indices, 0, social_table.shape[0] - 1)
    gi = jnp.clip(giver_indices, 0, giver_table.shape[0] - 1)
    shape3 = (NW, NCHUNK, CHUNK)
    ue, ie, se, ge = _sc_gather4(
        ui.reshape(shape3), ii.reshape(shape3),
        si.reshape(shape3), gi.reshape(shape3),
        user_table, item_table, social_table, giver_table)
    pred = _mlp(ue, ie, se, ge, W1, b1, W2, b2, W3, b3, W4, b4, Wo, bo)
    return pred.reshape(-1)
